# baseline (speedup 1.0000x reference)
import functools

import jax
import jax.numpy as jnp
from jax import lax
from jax.experimental import pallas as pl
from jax.experimental.pallas import tpu as pltpu
from jax.experimental.pallas import tpu_sc as plsc

_B, _S, _D = 4, 2048, 2048
_MAXLEN = 512
_NK, _KB = 8, 256
_PB = 64
_NPB = _MAXLEN // _PB
_NG = _NK

_NWORK = 32
_RW = (_B * _MAXLEN) // _NWORK


def _val_kernel(pinfo, x_ref, wa_ref, wb_ref, emb_ref, b_ref, out_ref,
                mean_ref, acc_ref):
    g = pl.program_id(0)

    @pl.when(g == 0)
    def _():
        acc_ref[...] = jnp.broadcast_to(b_ref[...][None, :], acc_ref.shape)

    ms = jnp.sum(x_ref[...], axis=1) * (1.0 / _S)
    mean_ref[:, pl.ds(g * _KB, _KB)] = ms
    ev = emb_ref[0, :, :]
    acc_ref[...] += jax.lax.dot_general(
        ms, wa_ref[...], (((1,), (1,)), ((), ())),
        preferred_element_type=jnp.float32)
    acc_ref[...] += jax.lax.dot_general(
        ev, wb_ref[...], (((1,), (1,)), ((), ())),
        preferred_element_type=jnp.float32)

    @pl.when(g == _NG - 1)
    def _():
        gate = jax.nn.sigmoid(acc_ref[...])
        out_ref[...] = gate * mean_ref[...]


_VAL_GRID = pltpu.PrefetchScalarGridSpec(
    num_scalar_prefetch=1,
    grid=(_NG,),
    in_specs=[
        pl.BlockSpec((_B, _S, _KB), lambda g, p: (0, 0, g)),
        pl.BlockSpec((_D, _KB), lambda g, p: (0, g)),
        pl.BlockSpec((_D, _KB), lambda g, p: (0, _NK + g)),
        pl.BlockSpec((1, 1, _KB), lambda g, p: (p[0], 0, g)),
        pl.BlockSpec((_D,), lambda g, p: (0,)),
    ],
    out_specs=pl.BlockSpec((_B, _D), lambda g, p: (0, 0)),
    scratch_shapes=[pltpu.VMEM((_B, _D), jnp.float32),
                    pltpu.VMEM((_B, _D), jnp.float32)],
)


_SC_MESH = plsc.VectorSubcoreMesh(core_axis_name="c", subcore_axis_name="s")


@functools.partial(
    pl.kernel, mesh=_SC_MESH,
    out_type=jax.ShapeDtypeStruct((_B * _MAXLEN, _D), jnp.float32))
def _sc_zero(zslab_hbm, out_hbm):
    w = lax.axis_index("s") * 2 + lax.axis_index("c")
    pltpu.sync_copy(zslab_hbm, out_hbm.at[pl.ds(w * _RW, _RW)])


def _combine_kernel(pinfo, bank_ref, val_ref, out_ref):
    pos = pinfo[0]
    out_ref[...] = jnp.zeros_like(out_ref)
    out_ref[:, pl.ds(pos % _PB, 1), :] = val_ref[...][:, None, :]


_COMBINE_GRID = pltpu.PrefetchScalarGridSpec(
    num_scalar_prefetch=1,
    grid=(1,),
    in_specs=[
        pl.BlockSpec(memory_space=pl.ANY),
        pl.BlockSpec((_B, _D), lambda g, p: (0, 0)),
    ],
    out_specs=pl.BlockSpec((_B, _PB, _D), lambda g, p: (0, p[0] // _PB, 0)),
)


@jax.jit
def _run(current_state, emb_table, W, b, pos):
    pinfo = jnp.reshape(pos, (1,))
    val = pl.pallas_call(
        _val_kernel,
        grid_spec=_VAL_GRID,
        out_shape=jax.ShapeDtypeStruct((_B, _D), jnp.float32),
        compiler_params=pltpu.CompilerParams(
            dimension_semantics=("arbitrary",)),
    )(pinfo, current_state, W, W, emb_table.reshape(_MAXLEN, 1, _D), b)

    zslab = jnp.zeros((_RW, _D), jnp.float32)
    bank = _sc_zero(zslab).reshape(_B, _MAXLEN, _D)

    return pl.pallas_call(
        _combine_kernel,
        grid_spec=_COMBINE_GRID,
        out_shape=jax.ShapeDtypeStruct((_B, _MAXLEN, _D), jnp.float32),
        input_output_aliases={1: 0},
        compiler_params=pltpu.CompilerParams(
            dimension_semantics=("arbitrary",)),
    )(pinfo, bank, val)


def kernel(current_state, emb_table, W, b, step):
    pos = jnp.asarray(step, jnp.int32) % _MAXLEN
    return _run(current_state, emb_table, W, b, pos)


# speedup vs baseline: 12.0793x; 12.0793x over previous
import jax
import jax.numpy as jnp
from jax.experimental import pallas as pl
from jax.experimental.pallas import tpu as pltpu

_B, _S, _D = 4, 2048, 2048
_MAXLEN = 512
_NK, _KB = 8, 256
_PB = _MAXLEN // _NK
_NG = _NK


def _scratch_kernel(pinfo, x1_ref, x2_ref, wa_ref, wb_ref, emb_ref, b_ref,
                    out_ref, mean_ref, acc_ref):
    g = pl.program_id(0)
    pos = pinfo[0]

    @pl.when(g == 0)
    def _():
        acc_ref[...] = jnp.broadcast_to(b_ref[...][None, :], acc_ref.shape)

    ms = jnp.concatenate(
        [jnp.sum(x1_ref[...], axis=1), jnp.sum(x2_ref[...], axis=1)],
        axis=0) * (1.0 / _S)
    mean_ref[:, pl.ds(g * _KB, _KB)] = ms
    ev = emb_ref[0, :, :]
    acc_ref[...] += jax.lax.dot_general(
        ms, wa_ref[...], (((1,), (1,)), ((), ())),
        preferred_element_type=jnp.float32)
    acc_ref[...] += jax.lax.dot_general(
        ev, wb_ref[...], (((1,), (1,)), ((), ())),
        preferred_element_type=jnp.float32)

    out_ref[...] = jnp.zeros_like(out_ref)

    @pl.when(g == _NG - 1)
    def _():
        gate = jax.nn.sigmoid(acc_ref[...])
        val = gate * mean_ref[...]
        out_ref[:, pl.ds(pos % _PB, 1), :] = val[:, None, :]


def _x1_map(g, pinfo):
    return (0, 0, g)


def _x2_map(g, pinfo):
    return (1, 0, g)


def _wa_map(g, pinfo):
    return (0, g)


def _wb_map(g, pinfo):
    return (0, _NK + g)


def _emb_map(g, pinfo):
    return (pinfo[0], 0, g)


def _b_map(g, pinfo):
    return (0,)


def _out_map(g, pinfo):
    pb = pinfo[0] // _PB
    return (0, (pb + 1 + g) % _NK, 0)


_GRID_SPEC = pltpu.PrefetchScalarGridSpec(
    num_scalar_prefetch=1,
    grid=(_NG,),
    in_specs=[
        pl.BlockSpec((_B // 2, _S, _KB), _x1_map),
        pl.BlockSpec((_B // 2, _S, _KB), _x2_map),
        pl.BlockSpec((_D, _KB), _wa_map),
        pl.BlockSpec((_D, _KB), _wb_map),
        pl.BlockSpec((1, 1, _KB), _emb_map),
        pl.BlockSpec((_D,), _b_map),
    ],
    out_specs=pl.BlockSpec((_B, _PB, _D), _out_map),
    scratch_shapes=[pltpu.VMEM((_B, _D), jnp.float32),
                    pltpu.VMEM((_B, _D), jnp.float32)],
)


@jax.jit
def _run(current_state, emb_table, W, b, pos):
    pinfo = jnp.reshape(pos, (1,))
    return pl.pallas_call(
        _scratch_kernel,
        grid_spec=_GRID_SPEC,
        out_shape=jax.ShapeDtypeStruct((_B, _MAXLEN, _D), jnp.float32),
        compiler_params=pltpu.CompilerParams(
            dimension_semantics=("arbitrary",)),
    )(pinfo, current_state, current_state, W, W,
      emb_table.reshape(_MAXLEN, 1, _D), b)


def kernel(current_state, emb_table, W, b, step):
    pos = jnp.asarray(step, jnp.int32) % _MAXLEN
    return _run(current_state, emb_table, W, b, pos)
